# baseline (speedup 1.0000x reference)
import functools

import jax
import jax.numpy as jnp
from jax import lax
from jax.experimental import pallas as pl
from jax.experimental.pallas import tpu as pltpu
from jax.experimental.pallas import tpu_sc as plsc


_TOK_BLK = 512
_LANES = 128


def _argmin_body(x_ref, cb_ref, idx_ref):
    xb = x_ref[...]
    cb = cb_ref[...]
    cross = lax.dot_general(
        xb, cb, (((1,), (1,)), ((), ())),
        preferred_element_type=jnp.float32)
    x_sq = jnp.sum(xb * xb, axis=1, keepdims=True)
    c_sq = jnp.sum(cb * cb, axis=1)
    d2 = (x_sq + c_sq[None, :]) - 2.0 * cross
    dist = jnp.sqrt(jnp.clip(d2, 0.0, None))
    k = dist.shape[1]
    ng = k // _LANES
    m = dist[:, :_LANES]
    gi = jnp.zeros_like(m)
    for g in range(1, ng):
        d = dist[:, g * _LANES:(g + 1) * _LANES]
        gi = jnp.where(d < m, jnp.float32(g), gi)
        m = jnp.minimum(m, d)
    mmin = jnp.min(m, axis=1, keepdims=True)
    lane = lax.broadcasted_iota(jnp.int32, m.shape, 1).astype(jnp.float32)
    fidx = gi * jnp.float32(_LANES) + lane
    idx_f = jnp.min(jnp.where(m == mmin, fidx, jnp.float32(k)), axis=1)
    idx_ref[0, 0, :] = idx_f.astype(jnp.int32)


def _nearest_indices(x2d, codebook, g0, grid):
    out = pl.pallas_call(
        _argmin_body,
        grid=(grid,),
        in_specs=[
            pl.BlockSpec((_TOK_BLK, x2d.shape[1]),
                         lambda i, g0=g0: (i + g0, 0)),
            pl.BlockSpec(codebook.shape, lambda i: (0, 0)),
        ],
        out_specs=pl.BlockSpec((1, 1, _TOK_BLK), lambda i: (i, 0, 0)),
        out_shape=jax.ShapeDtypeStruct((grid, 1, _TOK_BLK), jnp.int32),
    )(x2d, codebook)
    return out.reshape(grid * _TOK_BLK)


def _make_sc_gather(n_rows, d):
    info = plsc.get_sparse_core_info()
    nc, ns = info.num_cores, info.num_subcores
    nw = nc * ns
    per_w = n_rows // nw
    assert per_w * nw == n_rows and per_w <= 128 and per_w % 8 == 0
    mesh = plsc.VectorSubcoreMesh(core_axis_name="c", subcore_axis_name="s")

    @functools.partial(
        pl.kernel, mesh=mesh,
        out_type=jax.ShapeDtypeStruct((n_rows, d), jnp.float32),
        scratch_types=[
            pltpu.VMEM((per_w,), jnp.int32),
            pltpu.VMEM((per_w, d), jnp.float32),
            pltpu.SemaphoreType.DMA,
            pltpu.SemaphoreType.DMA,
        ],
    )
    def gather(table_hbm, idx_hbm, out_hbm, idx_v, rows_v, gsem, wsem):
        wid = lax.axis_index("s") * nc + lax.axis_index("c")
        pltpu.sync_copy(idx_hbm.at[wid], idx_v)
        g = pltpu.async_copy(table_hbm.at[idx_v], rows_v, gsem)
        g.wait()
        w = pltpu.async_copy(rows_v, out_hbm.at[pl.ds(wid * per_w, per_w)],
                             wsem)
        w.wait()

    return gather


_N_A = 2560
_sc_gather_a = _make_sc_gather(_N_A, 256)
_sc_gather_b = _make_sc_gather(4608 - _N_A, 256)


def kernel(x, codebook, embedding):
    b, t, d = x.shape
    x2d = x.reshape(b * t, d)
    idx_a = _nearest_indices(x2d, codebook, 0, _N_A // _TOK_BLK)
    out_a = _sc_gather_a(embedding, idx_a.reshape(32, -1))
    idx_b = _nearest_indices(x2d, codebook, _N_A // _TOK_BLK,
                             (b * t - _N_A) // _TOK_BLK)
    out_b = _sc_gather_b(embedding, idx_b.reshape(32, -1))
    return jnp.concatenate([out_a, out_b], axis=0).reshape(b, t, d)


# speedup vs baseline: 1.7106x; 1.7106x over previous
import jax
import jax.numpy as jnp
from jax import lax
from jax.experimental import pallas as pl


_TOK_BLK = 512
_LANES = 128


def _vq_body(x_ref, cb_ref, e1_ref, e2_ref, e3_ref, out_ref):
    xb = x_ref[...]
    cb = cb_ref[...]
    cross = lax.dot_general(
        xb, cb, (((1,), (1,)), ((), ())),
        preferred_element_type=jnp.float32)
    x_sq = jnp.sum(xb * xb, axis=1, keepdims=True)
    c_sq = jnp.sum(cb * cb, axis=1)
    d2 = (x_sq + c_sq[None, :]) - 2.0 * cross
    dist = jnp.sqrt(jnp.clip(d2, 0.0, None))
    k = dist.shape[1]
    ng = k // _LANES
    m = dist[:, :_LANES]
    gi = jnp.zeros_like(m)
    for g in range(1, ng):
        d = dist[:, g * _LANES:(g + 1) * _LANES]
        gi = jnp.where(d < m, jnp.float32(g), gi)
        m = jnp.minimum(m, d)
    mmin = jnp.min(m, axis=1, keepdims=True)
    lane = lax.broadcasted_iota(jnp.int32, m.shape, 1).astype(jnp.float32)
    fidx = gi * jnp.float32(_LANES) + lane
    idx_f = jnp.min(jnp.where(m == mmin, fidx, jnp.float32(k)), axis=1)
    kio = lax.broadcasted_iota(jnp.int32, cross.shape, 1).astype(jnp.float32)
    oh = jnp.where(kio == idx_f[:, None], jnp.float32(1),
                   jnp.float32(0)).astype(jnp.bfloat16)
    dn = (((1,), (0,)), ((), ()))
    g1 = lax.dot_general(oh, e1_ref[...], dn,
                         preferred_element_type=jnp.float32)
    g2 = lax.dot_general(oh, e2_ref[...], dn,
                         preferred_element_type=jnp.float32)
    g3 = lax.dot_general(oh, e3_ref[...], dn,
                         preferred_element_type=jnp.float32)
    out_ref[...] = g1 + (g2 + g3)


def kernel(x, codebook, embedding):
    b, t, d = x.shape
    n_tok = b * t
    x2d = x.reshape(n_tok, d)
    e1 = lax.optimization_barrier(embedding.astype(jnp.bfloat16))
    r1 = embedding - e1.astype(jnp.float32)
    e2 = lax.optimization_barrier(r1.astype(jnp.bfloat16))
    r2 = r1 - e2.astype(jnp.float32)
    e3 = lax.optimization_barrier(r2.astype(jnp.bfloat16))
    grid = n_tok // _TOK_BLK
    out = pl.pallas_call(
        _vq_body,
        grid=(grid,),
        in_specs=[
            pl.BlockSpec((_TOK_BLK, d), lambda i: (i, 0)),
            pl.BlockSpec(codebook.shape, lambda i: (0, 0)),
            pl.BlockSpec(embedding.shape, lambda i: (0, 0)),
            pl.BlockSpec(embedding.shape, lambda i: (0, 0)),
            pl.BlockSpec(embedding.shape, lambda i: (0, 0)),
        ],
        out_specs=pl.BlockSpec((_TOK_BLK, d), lambda i: (i, 0)),
        out_shape=jax.ShapeDtypeStruct((n_tok, d), jnp.float32),
    )(x2d, codebook, e1, e2, e3)
    return out.reshape(b, t, d)
